# Initial kernel scaffold; baseline (speedup 1.0000x reference)
import jax
import jax.numpy as jnp
from jax.experimental import pallas as pl


def kernel(visual_features, text_embeddings):
    raise NotImplementedError("write your pallas kernel here")


# speedup vs baseline: 1.6260x; 1.6260x over previous
import functools

import jax
import jax.numpy as jnp
from jax import lax
from jax.experimental import pallas as pl
from jax.experimental.pallas import tpu as pltpu

_B = 8
_R = 29
_K = 2048
_H = 768


def _region_body(q_ref, te_ref, sim_ref, idx_ref):
    q = q_ref[0]
    te = te_ref[0]
    qn = q / jnp.maximum(jnp.sqrt(jnp.sum(q * q, axis=1, keepdims=True)), 1e-12)
    tn = te / jnp.maximum(jnp.sqrt(jnp.sum(te * te, axis=1, keepdims=True)), 1e-12)
    sims = lax.dot_general(qn, tn, (((1,), (1,)), ((), ())),
                           preferred_element_type=jnp.float32)
    best = jnp.max(sims, axis=1)
    kiota = lax.broadcasted_iota(jnp.int32, (_B, _K), 1)
    bidx = jnp.min(jnp.where(sims == best[:, None], kiota, _K), axis=1)
    sim_ref[0, 0] = best
    idx_ref[0, 0] = bidx


@jax.jit
def _retrieve(vf_regions, text_embeddings):
    sim, idx = pl.pallas_call(
        _region_body,
        grid=(_R,),
        in_specs=[
            pl.BlockSpec((1, _B, _H), lambda r: (r, 0, 0)),
            pl.BlockSpec((1, _K, _H), lambda r: (r, 0, 0)),
        ],
        out_specs=[
            pl.BlockSpec((1, 1, _B), lambda r: (r, 0, 0)),
            pl.BlockSpec((1, 1, _B), lambda r: (r, 0, 0)),
        ],
        out_shape=[
            jax.ShapeDtypeStruct((_R, 1, _B), jnp.float32),
            jax.ShapeDtypeStruct((_R, 1, _B), jnp.int32),
        ],
        compiler_params=pltpu.CompilerParams(
            dimension_semantics=("arbitrary",),
        ),
    )(vf_regions, text_embeddings)
    return sim, idx


def kernel(visual_features, text_embeddings):
    vf_regions = jnp.transpose(visual_features[:, 1:1 + _R, :], (1, 0, 2))
    sim, idx = _retrieve(vf_regions, text_embeddings)
    best_sim = jnp.transpose(sim.reshape(_R, _B), (1, 0))
    best_idx = jnp.transpose(idx.reshape(_R, _B), (1, 0))
    return best_sim, best_idx
